# baseline (speedup 1.0000x reference)
import jax
import jax.numpy as jnp
from jax.experimental import pallas as pl

_IMG_H = 64
_IMG_W = 64
_NPIX = _IMG_H * _IMG_W
_N_CELLS = 64
_Z_WHAT = 32
_Z_WHERE = 4
_PER_CELL = 2 * _Z_WHAT + 2 * _Z_WHERE + 2 + 1
_SIGMA = 0.1
_PRIOR_PRES = 0.01
_SEQ_LEN = 10
_F = 5
_KTILE = 1024


def _enc_kernel(x_ref, w_ref, b_ref, out_ref):
    k = pl.program_id(0)
    acc = jnp.dot(x_ref[...], w_ref[...], preferred_element_type=jnp.float32)

    @pl.when(k == 0)
    def _():
        out_ref[...] = acc + b_ref[...]

    @pl.when(k > 0)
    def _():
        out_ref[...] += acc


def _kl_normal(mean, std):
    return 0.5 * (mean * mean + std * std - 2.0 * jnp.log(std) - 1.0)


def _dec_kernel(feat_ref, x_ref, act_ref, gw_ref, gb_ref, bgb_ref, pw_ref,
                eps_ref, y_ref, ll_ref, klw_ref, klwh_ref, kld_ref, klp_ref,
                cnt_ref):
    n = pl.program_id(0)
    f = feat_ref[...]

    start = jnp.maximum(n * _F - 1, 0)
    arows = act_ref[pl.ds(start, _F), :]
    shift = jnp.dot(arows, pw_ref[...],
                    preferred_element_type=jnp.float32)
    shift = jnp.where(n == 0, jnp.roll(shift, 1, axis=0), shift)
    i_iota = jax.lax.broadcasted_iota(jnp.int32, (_F, 1), 0)
    t0_mask = (i_iota == 0) & (n % 2 == 0)
    shift = jnp.where(t0_mask, 0.0, shift)

    what_mean = f[:, :, 0:_Z_WHAT] + shift[:, None, :]
    what_std = jax.nn.softplus(f[:, :, _Z_WHAT:2 * _Z_WHAT]) + 1e-4
    o = 2 * _Z_WHAT
    where_mean = f[:, :, o:o + _Z_WHERE]
    where_std = jax.nn.softplus(f[:, :, o + _Z_WHERE:o + 2 * _Z_WHERE]) + 1e-4
    o2 = o + 2 * _Z_WHERE
    depth_mean = f[:, :, o2:o2 + 1]
    depth_std = jax.nn.softplus(f[:, :, o2 + 1:o2 + 2]) + 1e-4
    pres_logit = f[:, :, o2 + 2:o2 + 3]
    z_pres = jax.nn.sigmoid(pres_logit)
    w_depth = jax.nn.sigmoid(-depth_mean)

    g = jax.nn.sigmoid(
        jnp.dot(what_mean.reshape(_F * _N_CELLS, _Z_WHAT), gw_ref[...],
                preferred_element_type=jnp.float32)
        + gb_ref[...])

    alpha = g[:, 3 * _NPIX:4 * _NPIX].reshape(_F, _N_CELLS, _NPIX) * z_pres
    imp = alpha * w_depth
    imp_sum = jnp.sum(imp, axis=1, keepdims=True)
    alpha_sum = jnp.clip(jnp.sum(alpha, axis=1, keepdims=True), 0.0, 1.0)
    denom = imp_sum + 1e-5
    bg = jax.nn.sigmoid(bgb_ref[...])

    ys = []
    for ch in range(3):
        gch = g[:, ch * _NPIX:(ch + 1) * _NPIX].reshape(_F, _N_CELLS, _NPIX)
        num = jnp.sum(gch * imp, axis=1, keepdims=True)
        ys.append(num / denom * alpha_sum
                  + bg[ch:ch + 1, :][None] * (1.0 - alpha_sum))
    y = jnp.concatenate(ys, axis=1)
    y_ref[...] = y

    x = x_ref[...]
    diff = (x - y) / _SIGMA
    ll = (-0.5 * jnp.sum(diff * diff, keepdims=True)
          + _F * 3.0 * _NPIX
          * (-jnp.log(_SIGMA) - 0.5 * jnp.log(2.0 * jnp.pi)))

    def _red(v):
        return jnp.sum(v, keepdims=True)

    kl_what = _red(_kl_normal(what_mean, what_std))
    kl_where = _red(_kl_normal(where_mean, where_std))
    kl_depth = _red(_kl_normal(depth_mean, depth_std))
    eps = eps_ref[0, 0]
    zp = jnp.clip(z_pres, eps, 1.0 - eps)
    kl_pres = _red(zp * jnp.log(zp / _PRIOR_PRES)
                   + (1.0 - zp) * jnp.log((1.0 - zp) / (1.0 - _PRIOR_PRES)))
    cnt_ref[...] = jnp.sum((z_pres > 0.7).astype(jnp.float32), axis=(1, 2),
                           keepdims=True)

    inv_t = 1.0 / _SEQ_LEN
    for ref, val in ((ll_ref, ll), (klw_ref, kl_what), (klwh_ref, kl_where),
                     (kld_ref, kl_depth), (klp_ref, kl_pres)):
        @pl.when(n % 2 == 0)
        def _(ref=ref, val=val):
            ref[...] = val * inv_t

        @pl.when(n % 2 == 1)
        def _(ref=ref, val=val):
            ref[...] += val * inv_t


def kernel(seq, actions, enc_W, enc_b, glimpse_W, glimpse_b, bg_W, bg_b,
           prop_W, eps):
    bs, seq_len = seq.shape[0], seq.shape[1]
    m = bs * seq_len
    kdim = 3 * _NPIX
    ndim = _N_CELLS * _PER_CELL

    x_flat = seq.reshape(m, kdim)
    feat = pl.pallas_call(
        _enc_kernel,
        grid=(kdim // _KTILE,),
        in_specs=[
            pl.BlockSpec((m, _KTILE), lambda k: (0, k)),
            pl.BlockSpec((_KTILE, ndim), lambda k: (k, 0)),
            pl.BlockSpec((1, ndim), lambda k: (0, 0)),
        ],
        out_specs=pl.BlockSpec((m, ndim), lambda k: (0, 0)),
        out_shape=jax.ShapeDtypeStruct((m, ndim), jnp.float32),
    )(x_flat, enc_W, enc_b.reshape(1, ndim))

    feat_r = feat.reshape(m, _N_CELLS, _PER_CELL)
    x_r = seq.reshape(m, 3, _NPIX)
    act_r = actions.reshape(m, actions.shape[-1])
    eps_arr = eps.reshape(1, 1)

    nsteps = m // _F
    scal_shape = jax.ShapeDtypeStruct((bs, 1, 1), jnp.float32)
    scal_spec = pl.BlockSpec((1, 1, 1), lambda n: (n // 2, 0, 0))
    outs = pl.pallas_call(
        _dec_kernel,
        grid=(nsteps,),
        in_specs=[
            pl.BlockSpec((_F, _N_CELLS, _PER_CELL), lambda n: (n, 0, 0)),
            pl.BlockSpec((_F, 3, _NPIX), lambda n: (n, 0, 0)),
            pl.BlockSpec((m, 4), lambda n: (0, 0)),
            pl.BlockSpec((_Z_WHAT, 4 * _NPIX), lambda n: (0, 0)),
            pl.BlockSpec((1, 4 * _NPIX), lambda n: (0, 0)),
            pl.BlockSpec((3, _NPIX), lambda n: (0, 0)),
            pl.BlockSpec((4, _Z_WHAT), lambda n: (0, 0)),
            pl.BlockSpec((1, 1), lambda n: (0, 0)),
        ],
        out_specs=[
            pl.BlockSpec((_F, 3, _NPIX), lambda n: (n, 0, 0)),
            scal_spec, scal_spec, scal_spec, scal_spec, scal_spec,
            pl.BlockSpec((_F, 1, 1), lambda n: (n, 0, 0)),
        ],
        out_shape=[
            jax.ShapeDtypeStruct((m, 3, _NPIX), jnp.float32),
            scal_shape, scal_shape, scal_shape, scal_shape, scal_shape,
            jax.ShapeDtypeStruct((m, 1, 1), jnp.float32),
        ],
    )(feat_r, x_r, act_r, glimpse_W, glimpse_b.reshape(1, 4 * _NPIX),
      bg_b.reshape(3, _NPIX), prop_W, eps_arr)
    y_flat, ll, klw, klwh, kld, klp, cnt = outs

    return (y_flat.reshape(bs, seq_len, 3, _IMG_H, _IMG_W),
            ll.reshape(bs),
            klw.reshape(bs),
            klwh.reshape(bs),
            kld.reshape(bs),
            klp.reshape(bs),
            cnt.reshape(bs, seq_len))
